# baseline (speedup 1.0000x reference)
import jax
import jax.numpy as jnp
from jax.experimental import pallas as pl
from jax.experimental.pallas import tpu as pltpu

N = 1024
H = 32
B = 8
T = 8
EMB = 10
EPAD = 16
OG = 2 * H
OU = H
KSV = 2 * H
NMISC = 3 * OG + 3 * OU


def _gcn_gru_body(x2t_ref, ep_ref, ept_ref,
                  wgt_ref, wut_ref, misc_ref, out_ref,
                  st_sc, state_sc, uall_sc, cz_sc, r_sc,
                  misc_sc):
    t = pl.program_id(0)

    @pl.when(t == 0)
    def _init():
        e = ep_ref[...]
        g = jnp.dot(e, ept_ref[...], preferred_element_type=jnp.float32)
        g = jnp.maximum(g, 0.0)
        m = jnp.max(g, axis=0, keepdims=True)
        ex = jnp.exp(g - m)
        s_t = ex / jnp.sum(ex, axis=0, keepdims=True)
        st_sc[...] = s_t.astype(jnp.bfloat16)
        misc_sc[...] = jnp.dot(misc_ref[...], ept_ref[...],
                               preferred_element_type=jnp.float32)
        uall_sc[...] = jnp.dot(x2t_ref[...], s_t,
                               preferred_element_type=jnp.float32)
        state_sc[...] = jnp.zeros_like(state_sc)

    s_t = st_sc[...]
    stt = state_sc[...]
    xr = x2t_ref[pl.ds(t * B, B)]
    ur = uall_sc[pl.ds(t * B, B)]
    misc = misc_sc[...]
    pxg = misc[0:OG]
    pug = misc[OG:2 * OG]
    bg = misc[2 * OG:3 * OG]
    pxu = misc[3 * OG:3 * OG + OU]
    puu = misc[3 * OG + OU:3 * OG + 2 * OU]
    bu = misc[3 * OG + 2 * OU:NMISC]

    v = jnp.dot(stt.astype(jnp.bfloat16), s_t,
                preferred_element_type=jnp.float32)

    et = ept_ref[...].astype(jnp.bfloat16)
    w2gt = wgt_ref[...]
    q_all = jnp.concatenate([
        jnp.concatenate(
            [et[d:d + 1] * jnp.concatenate(
                [stt[b * H:(b + 1) * H], v[b * H:(b + 1) * H]],
                axis=0).astype(jnp.bfloat16) for d in range(EMB)],
            axis=0) for b in range(B)], axis=1)
    lin_all = jnp.dot(w2gt, q_all, preferred_element_type=jnp.float32)
    for b in range(B):
        lin = lin_all[:, b * N:(b + 1) * N]
        acc = lin + bg + xr[b:b + 1] * pxg + ur[b:b + 1] * pug
        zr = jax.nn.sigmoid(acc)
        st_b = stt[b * H:(b + 1) * H]
        cz_sc[b * H:(b + 1) * H] = zr[:H] * st_b
        r_sc[b * H:(b + 1) * H] = zr[H:]

    cz = cz_sc[...]
    v2 = jnp.dot(cz.astype(jnp.bfloat16), s_t,
                 preferred_element_type=jnp.float32)

    w2ut = wut_ref[...]
    rr = r_sc[...]
    q2_all = jnp.concatenate([
        jnp.concatenate(
            [et[d:d + 1] * jnp.concatenate(
                [cz[b * H:(b + 1) * H], v2[b * H:(b + 1) * H]],
                axis=0).astype(jnp.bfloat16) for d in range(EMB)],
            axis=0) for b in range(B)], axis=1)
    lin2_all = jnp.dot(w2ut, q2_all, preferred_element_type=jnp.float32)
    for b in range(B):
        lin = lin2_all[:, b * N:(b + 1) * N]
        acc = lin + bu + xr[b:b + 1] * pxu + ur[b:b + 1] * puu
        hc = jnp.tanh(acc)
        r = rr[b * H:(b + 1) * H]
        ns = r * stt[b * H:(b + 1) * H] + (1.0 - r) * hc
        state_sc[b * H:(b + 1) * H] = ns
        out_ref[b, 0] = jnp.transpose(ns)


def kernel(x, node_embeddings, W_gate, b_gate, W_update, b_update):
    x0 = x[..., 0]
    x2t = jnp.transpose(x0, (1, 0, 2)).reshape(T * B, N)
    e_pad = jnp.pad(node_embeddings, ((0, 0), (0, EPAD - EMB)))
    e_pad_t = e_pad.T

    def _prep(wpool):
        o = wpool.shape[-1]
        wsv = jnp.concatenate([wpool[:, 0, 1:, :], wpool[:, 1, 1:, :]],
                              axis=1)
        return jnp.transpose(wsv, (2, 0, 1)).reshape(o, EMB * KSV)

    def _padr(a):
        return jnp.pad(a, ((0, 0), (0, EPAD - EMB)))

    wgt = _prep(W_gate).astype(jnp.bfloat16)
    wut = _prep(W_update).astype(jnp.bfloat16)
    misc = jnp.concatenate([
        _padr(W_gate[:, 0, 0, :].T),
        _padr(W_gate[:, 1, 0, :].T),
        _padr(b_gate.T),
        _padr(W_update[:, 0, 0, :].T),
        _padr(W_update[:, 1, 0, :].T),
        _padr(b_update.T),
    ], axis=0)
    const = lambda *_: (0, 0)
    out = pl.pallas_call(
        _gcn_gru_body,
        grid=(T,),
        in_specs=[
            pl.BlockSpec((T * B, N), const),
            pl.BlockSpec((N, EPAD), const),
            pl.BlockSpec((EPAD, N), const),
            pl.BlockSpec((OG, EMB * KSV), const),
            pl.BlockSpec((OU, EMB * KSV), const),
            pl.BlockSpec((NMISC, EPAD), const),
        ],
        out_specs=pl.BlockSpec((B, 1, N, H), lambda t: (0, t, 0, 0)),
        out_shape=jax.ShapeDtypeStruct((B, T, N, H), jnp.float32),
        scratch_shapes=[
            pltpu.VMEM((N, N), jnp.bfloat16),
            pltpu.VMEM((B * H, N), jnp.float32),
            pltpu.VMEM((T * B, N), jnp.float32),
            pltpu.VMEM((B * H, N), jnp.float32),
            pltpu.VMEM((B * H, N), jnp.float32),
            pltpu.VMEM((NMISC, N), jnp.float32),
        ],
    )(x2t, e_pad, e_pad_t, wgt, wut, misc)

    return (out, out[:, -1])


# speedup vs baseline: 1.0429x; 1.0429x over previous
import jax
import jax.numpy as jnp
from jax.experimental import pallas as pl
from jax.experimental.pallas import tpu as pltpu

N = 1024
H = 32
B = 8
T = 8
EMB = 10
EPAD = 16
OG = 2 * H
OU = H
KSV = 2 * H
NMISC = 3 * OG + 3 * OU


def _gcn_gru_body(x2t_ref, ep_ref, ept_ref,
                  wgt_ref, wut_ref, misc_ref, out_ref,
                  st_sc, state_sc, uall_sc, cz_sc, r_sc,
                  misc_sc):
    t = pl.program_id(0)

    @pl.when(t == 0)
    def _init():
        e = ep_ref[...]
        g = jnp.dot(e, ept_ref[...], preferred_element_type=jnp.float32)
        g = jnp.maximum(g, 0.0)
        m = jnp.max(g, axis=0, keepdims=True)
        ex = jnp.exp(g - m)
        s_t = ex / jnp.sum(ex, axis=0, keepdims=True)
        st_sc[...] = s_t.astype(jnp.bfloat16)
        misc_sc[...] = jnp.dot(misc_ref[...], ept_ref[...],
                               preferred_element_type=jnp.float32)
        uall_sc[...] = jnp.dot(x2t_ref[...], s_t,
                               preferred_element_type=jnp.float32)
        state_sc[...] = jnp.zeros_like(state_sc)

    s_t = st_sc[...]
    stt = state_sc[...]
    xr = x2t_ref[pl.ds(t * B, B)]
    ur = uall_sc[pl.ds(t * B, B)]
    misc = misc_sc[...]
    pxg = misc[0:OG]
    pug = misc[OG:2 * OG]
    bg = misc[2 * OG:3 * OG]
    pxu = misc[3 * OG:3 * OG + OU]
    puu = misc[3 * OG + OU:3 * OG + 2 * OU]
    bu = misc[3 * OG + 2 * OU:NMISC]

    v = jnp.dot(stt.astype(jnp.bfloat16), s_t,
                preferred_element_type=jnp.float32)

    et = ept_ref[...].astype(jnp.bfloat16)
    w2gt = wgt_ref[...]
    for b in range(B):
        st_b = stt[b * H:(b + 1) * H]
        sv = jnp.concatenate([st_b, v[b * H:(b + 1) * H]],
                             axis=0).astype(jnp.bfloat16)
        q = jnp.concatenate(
            [et[d:d + 1] * sv for d in range(EMB)],
            axis=0)
        lin = jnp.dot(w2gt, q, preferred_element_type=jnp.float32)
        acc = lin + bg + xr[b:b + 1] * pxg + ur[b:b + 1] * pug
        zr = jax.nn.sigmoid(acc)
        cz_sc[b * H:(b + 1) * H] = zr[:H] * st_b
        r_sc[b * H:(b + 1) * H] = zr[H:]

    cz = cz_sc[...]
    v2 = jnp.dot(cz.astype(jnp.bfloat16), s_t,
                 preferred_element_type=jnp.float32)

    w2ut = wut_ref[...]
    rr = r_sc[...]
    for b in range(B):
        sv = jnp.concatenate([cz[b * H:(b + 1) * H], v2[b * H:(b + 1) * H]],
                             axis=0).astype(jnp.bfloat16)
        q = jnp.concatenate(
            [et[d:d + 1] * sv for d in range(EMB)],
            axis=0)
        lin = jnp.dot(w2ut, q, preferred_element_type=jnp.float32)
        acc = lin + bu + xr[b:b + 1] * pxu + ur[b:b + 1] * puu
        hc = jnp.tanh(acc)
        r = rr[b * H:(b + 1) * H]
        ns = r * stt[b * H:(b + 1) * H] + (1.0 - r) * hc
        state_sc[b * H:(b + 1) * H] = ns
        out_ref[0, b * H:(b + 1) * H, :] = ns


def kernel(x, node_embeddings, W_gate, b_gate, W_update, b_update):
    x0 = x[..., 0]
    x2t = jnp.transpose(x0, (1, 0, 2)).reshape(T * B, N)
    e_pad = jnp.pad(node_embeddings, ((0, 0), (0, EPAD - EMB)))
    e_pad_t = e_pad.T

    def _prep(wpool):
        o = wpool.shape[-1]
        wsv = jnp.concatenate([wpool[:, 0, 1:, :], wpool[:, 1, 1:, :]],
                              axis=1)
        return jnp.transpose(wsv, (2, 0, 1)).reshape(o, EMB * KSV)

    def _padr(a):
        return jnp.pad(a, ((0, 0), (0, EPAD - EMB)))

    wgt = _prep(W_gate).astype(jnp.bfloat16)
    wut = _prep(W_update).astype(jnp.bfloat16)
    misc = jnp.concatenate([
        _padr(W_gate[:, 0, 0, :].T),
        _padr(W_gate[:, 1, 0, :].T),
        _padr(b_gate.T),
        _padr(W_update[:, 0, 0, :].T),
        _padr(W_update[:, 1, 0, :].T),
        _padr(b_update.T),
    ], axis=0)
    const = lambda *_: (0, 0)
    out = pl.pallas_call(
        _gcn_gru_body,
        grid=(T,),
        in_specs=[
            pl.BlockSpec((T * B, N), const),
            pl.BlockSpec((N, EPAD), const),
            pl.BlockSpec((EPAD, N), const),
            pl.BlockSpec((OG, EMB * KSV), const),
            pl.BlockSpec((OU, EMB * KSV), const),
            pl.BlockSpec((NMISC, EPAD), const),
        ],
        out_specs=pl.BlockSpec((1, B * H, N), lambda t: (t, 0, 0)),
        out_shape=jax.ShapeDtypeStruct((T, B * H, N), jnp.float32),
        scratch_shapes=[
            pltpu.VMEM((N, N), jnp.bfloat16),
            pltpu.VMEM((B * H, N), jnp.float32),
            pltpu.VMEM((T * B, N), jnp.float32),
            pltpu.VMEM((B * H, N), jnp.float32),
            pltpu.VMEM((B * H, N), jnp.float32),
            pltpu.VMEM((NMISC, N), jnp.float32),
        ],
    )(x2t, e_pad, e_pad_t, wgt, wut, misc)

    layer_output = jnp.transpose(out.reshape(T, B, H, N), (1, 0, 3, 2))
    return (layer_output, layer_output[:, -1])
